# Initial kernel scaffold; baseline (speedup 1.0000x reference)
import jax
import jax.numpy as jnp
from jax.experimental import pallas as pl


def kernel(nodes, neighbors, emb_table, W1, b1, gamma, beta):
    raise NotImplementedError("write your pallas kernel here")


# speedup vs baseline: 8.6164x; 8.6164x over previous
import functools

import jax
import jax.numpy as jnp
from jax import lax
from jax.experimental import pallas as pl
from jax.experimental.pallas import tpu as pltpu
from jax.experimental.pallas import tpu_sc as plsc

B = 16384
DEG = 32
D = 128
NC = 2
NS = 16
NW = NC * NS
BPW = B // NW
CH = 128
NPC = CH // DEG
NCHUNK = BPW * DEG // CH


def _sc_gather(table, neigh_flat, nodes, zeros):
    mesh = plsc.VectorSubcoreMesh(core_axis_name="c", subcore_axis_name="s")

    @functools.partial(
        pl.kernel,
        mesh=mesh,
        out_type=[
            jax.ShapeDtypeStruct((B, D), jnp.float32),
            jax.ShapeDtypeStruct((B, D), jnp.float32),
        ],
        scratch_types=[
            pltpu.VMEM((BPW * DEG,), jnp.int32),
            pltpu.VMEM((BPW,), jnp.int32),
            pltpu.VMEM((2, CH, D), jnp.float32),
            pltpu.VMEM((CH,), jnp.int32),
            pltpu.VMEM_SHARED((NS * BPW, D), jnp.float32),
            pltpu.SemaphoreType.DMA((2,)),
        ],
    )
    def k(table_h, gidx_h, nidx_h, zeros_h, self_o, nsum_o,
          gidx, nidx, bufs, scat, acc, gsem):
        c = lax.axis_index("c")
        s = lax.axis_index("s")
        base = (c * NS + s) * BPW

        pltpu.sync_copy(gidx_h.at[pl.ds(base * DEG, BPW * DEG)], gidx)
        pltpu.sync_copy(nidx_h.at[pl.ds(base, BPW)], nidx)
        pltpu.sync_copy(zeros_h, acc.at[pl.ds(s * BPW, BPW)])

        def gcopy(ci, b):
            off = pl.multiple_of(ci * CH, CH)
            return pltpu.make_async_copy(
                table_h.at[gidx.at[pl.ds(off, CH)]], bufs.at[b], gsem.at[b])

        def scadd(ci, b):
            segbase = s * BPW + ci * NPC
            for l in range(CH // 16):
                scat[pl.ds(l * 16, 16)] = (
                    jnp.zeros((16,), jnp.int32) + (segbase + l * 16 // DEG))
            pltpu.sync_copy(bufs.at[b], acc.at[scat], add=True)

        gcopy(0, 0).start()

        def body(i, carry):
            c0 = 2 * i
            gcopy(c0 + 1, 1).start()
            gcopy(c0, 0).wait()
            scadd(c0, 0)

            @pl.when(i < NCHUNK // 2 - 1)
            def _():
                gcopy(c0 + 2, 0).start()

            gcopy(c0 + 1, 1).wait()
            scadd(c0 + 1, 1)
            return carry

        lax.fori_loop(0, NCHUNK // 2, body, 0)

        def sget(kk, b):
            return pltpu.make_async_copy(
                table_h.at[nidx.at[pl.ds(kk * CH, CH)]], bufs.at[b], gsem.at[b])

        sget(0, 0).start()
        for kk in range(BPW // CH):
            if kk + 1 < BPW // CH:
                sget(kk + 1, (kk + 1) % 2).start()
            sget(kk, kk % 2).wait()
            dst = pl.multiple_of(base + kk * CH, CH)
            pltpu.sync_copy(bufs.at[kk % 2], self_o.at[pl.ds(dst, CH)])

        pltpu.sync_copy(acc.at[pl.ds(s * BPW, BPW)],
                        nsum_o.at[pl.ds(pl.multiple_of(base, CH), BPW)])

    return k(table, neigh_flat, nodes, zeros)


def _tc_dense(self_feats, nsum, W1, b1, gamma, beta):
    def body(x_ref, n_ref, w_ref, b_ref, g_ref, bb_ref, o_ref):
        x = x_ref[...]
        n = n_ref[...] * (1.0 / DEG)
        w = w_ref[...]
        lin = lax.dot_general(x, w[:, :D], (((1,), (1,)), ((), ())),
                              preferred_element_type=jnp.float32)
        lin = lin + lax.dot_general(n, w[:, D:], (((1,), (1,)), ((), ())),
                                    preferred_element_type=jnp.float32)
        lin = lin + b_ref[...]
        mu = jnp.mean(lin, axis=0, keepdims=True)
        xc = lin - mu
        var = jnp.mean(xc * xc, axis=0, keepdims=True)
        y = xc * lax.rsqrt(var + 1e-5) * g_ref[...] + bb_ref[...]
        o_ref[...] = jnp.maximum(y, 0.0)

    return pl.pallas_call(
        body,
        out_shape=jax.ShapeDtypeStruct((B, D), jnp.float32),
    )(self_feats, nsum, W1,
      b1.reshape(1, D), gamma.reshape(1, D), beta.reshape(1, D))


def kernel(nodes, neighbors, emb_table, W1, b1, gamma, beta):
    zeros = jnp.zeros((BPW, D), jnp.float32)
    self_feats, nsum = _sc_gather(emb_table, neighbors.reshape(-1), nodes, zeros)
    return _tc_dense(self_feats, nsum, W1, b1, gamma, beta)
